# Initial kernel scaffold; baseline (speedup 1.0000x reference)
import jax
import jax.numpy as jnp
from jax.experimental import pallas as pl


def kernel(x, adj):
    raise NotImplementedError("write your pallas kernel here")


# speedup vs baseline: 2.0305x; 2.0305x over previous
import jax
import jax.numpy as jnp
from jax.experimental import pallas as pl
from jax.experimental.pallas import tpu as pltpu

EPS = 1e-7
MAX_NORM = 1.0 - 1e-5


def _logmap0_body(x_ref, o_ref):
    x = x_ref[...]
    norm = jnp.clip(jnp.sqrt(jnp.sum(x * x, axis=-1, keepdims=True)), EPS, None)
    z = jnp.clip(norm, None, MAX_NORM)
    atanh = 0.5 * jnp.log((1.0 + z) / (1.0 - z))
    o_ref[...] = atanh * x / norm


def _agg_body(adj_ref, xt_ref, o_ref):
    blk = adj_ref[...]
    xt = xt_ref[...]
    acc = jnp.dot(blk, xt, preferred_element_type=jnp.float32)
    cnt = jnp.sum(blk, axis=1, keepdims=True)
    agg = acc / cnt
    norm = jnp.clip(jnp.sqrt(jnp.sum(agg * agg, axis=-1, keepdims=True)), EPS, None)
    res = jnp.tanh(norm) * agg / norm
    norm2 = jnp.clip(jnp.sqrt(jnp.sum(res * res, axis=-1, keepdims=True)), EPS, None)
    o_ref[...] = res * jnp.minimum(1.0, MAX_NORM / norm2)


def kernel(x, adj):
    N, D = x.shape
    xt = pl.pallas_call(
        _logmap0_body,
        grid=(5,),
        in_specs=[pl.BlockSpec((N // 5, D), lambda i: (i, 0))],
        out_specs=pl.BlockSpec((N // 5, D), lambda i: (i, 0)),
        out_shape=jax.ShapeDtypeStruct((N, D), jnp.float32),
    )(x)

    BM = 400
    return pl.pallas_call(
        _agg_body,
        grid=(N // BM,),
        in_specs=[
            pl.BlockSpec((BM, N), lambda i: (i, 0)),
            pl.BlockSpec((N, D), lambda i: (0, 0)),
        ],
        out_specs=pl.BlockSpec((BM, D), lambda i: (i, 0)),
        out_shape=jax.ShapeDtypeStruct((N, D), jnp.float32),
        compiler_params=pltpu.CompilerParams(
            dimension_semantics=("parallel",),
        ),
    )(adj, xt)
